# Initial kernel scaffold; baseline (speedup 1.0000x reference)
import jax
import jax.numpy as jnp
from jax.experimental import pallas as pl


def kernel(rows, cols, cos_sin):
    raise NotImplementedError("write your pallas kernel here")


# speedup vs baseline: 1.1376x; 1.1376x over previous
import functools

import jax
import jax.numpy as jnp
from jax import lax
from jax.experimental import pallas as pl
from jax.experimental.pallas import tpu as pltpu
from jax.experimental.pallas import tpu_sc as plsc

D = 2048
B = 2 * 8192
NC = 2
NS = 16
NW = NC * NS
B_PER_W = B // NW
CH = 32
N_CHUNKS = B_PER_W // CH


def _gather_body(table_hbm, idx_hbm, out_hbm, idx_v, buf, gsem):
    wid = lax.axis_index("s") * NC + lax.axis_index("c")
    base = wid * B_PER_W
    pltpu.sync_copy(idx_hbm.at[pl.ds(base, B_PER_W)], idx_v)

    def chunk_fn(g, carry):
        cp = pltpu.async_copy(
            table_hbm.at[idx_v.at[pl.ds(g * CH, CH)]], buf, gsem)
        cp.wait()
        pltpu.sync_copy(buf, out_hbm.at[pl.ds(base + g * CH, CH)])
        return carry

    lax.fori_loop(0, N_CHUNKS, chunk_fn, 0)


@jax.jit
def kernel(rows, cols, cos_sin):
    idx = jnp.stack([rows[0], cols[0]], axis=1).reshape(B)
    mesh = plsc.VectorSubcoreMesh(core_axis_name="c", subcore_axis_name="s")
    gathered = pl.kernel(
        _gather_body,
        mesh=mesh,
        out_type=jax.ShapeDtypeStruct((B, D), jnp.float32),
        scratch_types=[
            pltpu.VMEM((B_PER_W,), jnp.int32),
            pltpu.VMEM((CH, D), jnp.float32),
            pltpu.SemaphoreType.DMA,
        ],
    )(cos_sin, idx)
    return gathered.reshape(1, 8192, 4096)


# speedup vs baseline: 1.1782x; 1.0357x over previous
import functools

import jax
import jax.numpy as jnp
from jax import lax
from jax.experimental import pallas as pl
from jax.experimental.pallas import tpu as pltpu
from jax.experimental.pallas import tpu_sc as plsc

D = 2048
B = 2 * 8192
NC = 2
NS = 16
NW = NC * NS
B_PER_W = B // NW
CH = 16
NBUF = 2
N_CHUNKS = B_PER_W // CH
N_ROUNDS = N_CHUNKS // NBUF


def _gather_body(table_hbm, idx_hbm, out_hbm, idx_v, bufs, gsem, wsem):
    wid = lax.axis_index("s") * NC + lax.axis_index("c")
    base = wid * B_PER_W
    pltpu.sync_copy(idx_hbm.at[pl.ds(base, B_PER_W)], idx_v)

    def start_gather(g, b):
        pltpu.async_copy(
            table_hbm.at[idx_v.at[pl.ds(g * CH, CH)]], bufs.at[b],
            gsem.at[b])

    def wait_gather(b):
        pltpu.make_async_copy(
            table_hbm.at[idx_v.at[pl.ds(0, CH)]], bufs.at[b],
            gsem.at[b]).wait()

    for b in range(NBUF):
        start_gather(b, b)

    def round_fn(r, carry):
        for b in range(NBUF):
            g = r * NBUF + b
            wait_gather(b)
            cp = pltpu.make_async_copy(
                bufs.at[b], out_hbm.at[pl.ds(base + g * CH, CH)], wsem.at[b])
            cp.start()
            cp.wait()

            @pl.when(g + NBUF < N_CHUNKS)
            def _():
                start_gather(g + NBUF, b)
        return carry

    lax.fori_loop(0, N_ROUNDS, round_fn, 0)


@jax.jit
def kernel(rows, cols, cos_sin):
    idx = jnp.stack([rows[0], cols[0]], axis=1).reshape(B)
    mesh = plsc.VectorSubcoreMesh(core_axis_name="c", subcore_axis_name="s")
    gathered = pl.kernel(
        _gather_body,
        mesh=mesh,
        out_type=jax.ShapeDtypeStruct((B, D), jnp.float32),
        scratch_types=[
            pltpu.VMEM((B_PER_W,), jnp.int32),
            pltpu.VMEM((NBUF, CH, D), jnp.float32),
            pltpu.SemaphoreType.DMA((NBUF,)),
            pltpu.SemaphoreType.DMA((NBUF,)),
        ],
    )(cos_sin, idx)
    return gathered.reshape(1, 8192, 4096)


# speedup vs baseline: 2.7087x; 2.2989x over previous
import jax
import jax.numpy as jnp
from jax import lax
from jax.experimental import pallas as pl
from jax.experimental.pallas import tpu as pltpu
from jax.experimental.pallas import tpu_sc as plsc

D = 2048
R = 8192
NC = 2
NS = 16
NW = NC * NS
R_PER_W = R // NW
CH2 = 8
NBUF = 2
N_CHUNKS = R_PER_W // CH2
N_ROUNDS = N_CHUNKS // NBUF


def _gather_body(table_hbm, idx_hbm, out_hbm, idx_v, bufs, gsem, wsem):
    wid = lax.axis_index("s") * NC + lax.axis_index("c")
    base = wid * R_PER_W
    pltpu.sync_copy(idx_hbm.at[pl.ds(base, R_PER_W)], idx_v.at[pl.ds(0, R_PER_W)])
    pltpu.sync_copy(idx_hbm.at[pl.ds(R + base, R_PER_W)],
                    idx_v.at[pl.ds(R_PER_W, R_PER_W)])

    def start_gathers(g, b):
        pltpu.async_copy(
            table_hbm.at[idx_v.at[pl.ds(g * CH2, CH2)]],
            bufs.at[b, pl.ds(0, CH2), pl.ds(0, D)], gsem.at[b])
        pltpu.async_copy(
            table_hbm.at[idx_v.at[pl.ds(R_PER_W + g * CH2, CH2)]],
            bufs.at[b, pl.ds(0, CH2), pl.ds(D, D)], gsem.at[b])

    def wait_gathers(b):
        pltpu.make_async_copy(
            table_hbm.at[idx_v.at[pl.ds(0, CH2)]],
            bufs.at[b, pl.ds(0, CH2), pl.ds(0, D)], gsem.at[b]).wait()
        pltpu.make_async_copy(
            table_hbm.at[idx_v.at[pl.ds(0, CH2)]],
            bufs.at[b, pl.ds(0, CH2), pl.ds(D, D)], gsem.at[b]).wait()

    for b in range(NBUF):
        start_gathers(b, b)

    def round_fn(r, carry):
        for b in range(NBUF):
            g = r * NBUF + b
            wait_gathers(b)
            cp = pltpu.make_async_copy(
                bufs.at[b], out_hbm.at[0, pl.ds(base + g * CH2, CH2)],
                wsem.at[b])
            cp.start()
            cp.wait()

            @pl.when(g + NBUF < N_CHUNKS)
            def _():
                start_gathers(g + NBUF, b)
        return carry

    lax.fori_loop(0, N_ROUNDS, round_fn, 0)


@jax.jit
def kernel(rows, cols, cos_sin):
    idx = jnp.concatenate([rows[0], cols[0]])
    mesh = plsc.VectorSubcoreMesh(core_axis_name="c", subcore_axis_name="s")
    return pl.kernel(
        _gather_body,
        mesh=mesh,
        out_type=jax.ShapeDtypeStruct((1, R, 2 * D), jnp.float32),
        scratch_types=[
            pltpu.VMEM((2 * R_PER_W,), jnp.int32),
            pltpu.VMEM((NBUF, CH2, 2 * D), jnp.float32),
            pltpu.SemaphoreType.DMA((NBUF,)),
            pltpu.SemaphoreType.DMA((NBUF,)),
        ],
    )(cos_sin, idx)


# speedup vs baseline: 2.7131x; 1.0016x over previous
import jax
import jax.numpy as jnp
from jax import lax
from jax.experimental import pallas as pl
from jax.experimental.pallas import tpu as pltpu
from jax.experimental.pallas import tpu_sc as plsc

D = 2048
R = 8192
NC = 2
NS = 16
NW = NC * NS
R_PER_W = R // NW
CH2 = 8
NBUF = 3
N_CHUNKS = R_PER_W // CH2
N_ROUNDS = (N_CHUNKS + NBUF - 1) // NBUF


def _gather_body(table_hbm, rows_hbm, cols_hbm, out_hbm, idx_v, bufs, gsem,
                 wsem):
    wid = lax.axis_index("s") * NC + lax.axis_index("c")
    base = wid * R_PER_W
    pltpu.sync_copy(rows_hbm.at[0, pl.ds(base, R_PER_W)],
                    idx_v.at[pl.ds(0, R_PER_W)])
    pltpu.sync_copy(cols_hbm.at[0, pl.ds(base, R_PER_W)],
                    idx_v.at[pl.ds(R_PER_W, R_PER_W)])

    def start_gathers(g, b):
        pltpu.async_copy(
            table_hbm.at[idx_v.at[pl.ds(g * CH2, CH2)]],
            bufs.at[b, pl.ds(0, CH2), pl.ds(0, D)], gsem.at[b])
        pltpu.async_copy(
            table_hbm.at[idx_v.at[pl.ds(R_PER_W + g * CH2, CH2)]],
            bufs.at[b, pl.ds(0, CH2), pl.ds(D, D)], gsem.at[b])

    def wait_gathers(b):
        pltpu.make_async_copy(
            out_hbm.at[0, pl.ds(0, CH2)], bufs.at[b], gsem.at[b]).wait()

    for b in range(NBUF):
        start_gathers(b, b)

    def round_fn(r, carry):
        for b in range(NBUF):
            g = r * NBUF + b

            @pl.when(g < N_CHUNKS)
            def _():
                wait_gathers(b)
                cp = pltpu.make_async_copy(
                    bufs.at[b], out_hbm.at[0, pl.ds(base + g * CH2, CH2)],
                    wsem.at[b])
                cp.start()
                cp.wait()

                @pl.when(g + NBUF < N_CHUNKS)
                def _():
                    start_gathers(g + NBUF, b)
        return carry

    lax.fori_loop(0, N_ROUNDS, round_fn, 0)


@jax.jit
def kernel(rows, cols, cos_sin):
    mesh = plsc.VectorSubcoreMesh(core_axis_name="c", subcore_axis_name="s")
    return pl.kernel(
        _gather_body,
        mesh=mesh,
        out_type=jax.ShapeDtypeStruct((1, R, 2 * D), jnp.float32),
        scratch_types=[
            pltpu.VMEM((2 * R_PER_W,), jnp.int32),
            pltpu.VMEM((NBUF, CH2, 2 * D), jnp.float32),
            pltpu.SemaphoreType.DMA((NBUF,)),
            pltpu.SemaphoreType.DMA((NBUF,)),
        ],
    )(cos_sin, rows, cols)
